# baseline (speedup 1.0000x reference)
import functools

import jax
import jax.numpy as jnp
from jax import lax
from jax.experimental import pallas as pl
from jax.experimental.pallas import tpu as pltpu
from jax.experimental.pallas import tpu_sc as plsc

N = 10000
D = 128
NHEADS1 = 8
E = 320000
ETOT = E + N

NC = 2
NS = 16
NW = NC * NS

NPAD = 10112
ROWS_PER_TILE = NPAD // NS

TCOLS = 144
ACOLS = 16

C = 40
NCHUNK = 264
PER_W = C * NCHUNK
EPAD = PER_W * NW
NROT = 6
NIDX = 12
LOOK = 4

_SENT = -1e30


def _head_expand_mask(heads, oc):
    r = lax.broadcasted_iota(jnp.int32, (heads, 128), 0)
    c = lax.broadcasted_iota(jnp.int32, (heads, 128), 1) // oc
    return (r == c).astype(jnp.float32)


@functools.lru_cache(maxsize=None)
def _make_edge_kernel(heads):
    oc = 128 // heads
    mesh = plsc.VectorSubcoreMesh(core_axis_name="c", subcore_axis_name="s",
                                  num_cores=NC, num_subcores=NS)

    @functools.partial(
        pl.kernel,
        mesh=mesh,
        compiler_params=pltpu.CompilerParams(use_tc_tiling_on_sc=False,
                                             needs_layout_passes=False),
        out_type=jax.ShapeDtypeStruct((NC, NPAD, TCOLS), jnp.float32),
        scratch_types=[
            pltpu.VMEM((NIDX, 2, C), jnp.int32),
            pltpu.VMEM((NROT, C, TCOLS), jnp.float32),
            pltpu.VMEM((NROT, C, ACOLS), jnp.float32),
            pltpu.VMEM_SHARED((NPAD, TCOLS), jnp.float32),
            pltpu.SemaphoreType.DMA((NIDX,)),
            pltpu.SemaphoreType.DMA((NROT,)),
            pltpu.SemaphoreType.DMA((NROT,)),
        ],
    )
    def edge_kernel(tab_hbm, adst_hbm, ed_hbm, out_hbm,
                    edb, rows, adstb, acc, semi, semg, sems):
        cid = lax.axis_index("c")
        sid = lax.axis_index("s")
        wid = cid * NS + sid

        def _zrow(i, _):
            for k in range(TCOLS // 16):
                rows[0, i, pl.ds(k * 16, 16)] = jnp.zeros((16,), jnp.float32)
            return 0
        lax.fori_loop(0, C, _zrow, 0)
        r0 = sid * ROWS_PER_TILE
        for off in range(0, ROWS_PER_TILE, C):
            nrow = min(C, ROWS_PER_TILE - off)
            pltpu.sync_copy(rows.at[0, pl.ds(0, nrow)],
                            acc.at[pl.ds(r0 + off, nrow)])
        plsc.subcore_barrier()

        iota16 = lax.iota(jnp.int32, 16)
        ebase = wid * PER_W

        def fire_idx(ch, k):
            pltpu.async_copy(ed_hbm.at[:, pl.ds(ebase + ch * C, C)],
                             edb.at[k], semi.at[k])

        def wait_idx(ch, k):
            pltpu.make_async_copy(ed_hbm.at[:, pl.ds(ebase + ch * C, C)],
                                  edb.at[k], semi.at[k]).wait()

        def fire_gather(r, k):
            pltpu.async_copy(tab_hbm.at[edb.at[k, 0]], rows.at[r], semg.at[r])

        def wait_gather(r, k):
            pltpu.make_async_copy(tab_hbm.at[edb.at[k, 0]], rows.at[r],
                                  semg.at[r]).wait()

        def fire_scatter(r, k):
            pass

        def wait_scatter(r, k):
            pass

        def compute(r):
            @plsc.parallel_loop(0, C // 16, unroll=2)
            def jbody(j):
                rb = j * 16 + iota16
                for h in range(heads):
                    colv = jnp.full((16,), 128 + h, jnp.int32)
                    av = plsc.load_gather(rows.at[r], [rb, colv])
                    dv = plsc.load_gather(adstb.at[r],
                                          [rb, jnp.full((16,), h, jnp.int32)])
                    a = av + dv
                    a = jnp.maximum(a, a * jnp.float32(0.2))
                    plsc.store_scatter(rows.at[r], [rb, colv], jnp.exp(a))

            @plsc.parallel_loop(0, C, unroll=4)
            def ebody(e):
                exv = rows[r, e, pl.ds(128, 16)]
                for h in range(heads):
                    s = exv[h]
                    for cc in range(oc // 16):
                        col = h * oc + cc * 16
                        rows[r, e, pl.ds(col, 16)] = \
                            rows[r, e, pl.ds(col, 16)] * s

        for i in range(2 * LOOK):
            fire_idx(i, i % NIDX)
        for i in range(LOOK):
            wait_idx(i, i % NIDX)
            fire_gather(i % NROT, i % NIDX)

        def block_body(t, _):
            ch0 = t * 12
            for k12 in range(12):
                ch = ch0 + k12
                r = k12 % NROT
                k = k12 % NIDX
                r4 = (k12 + LOOK) % NROT
                k4 = (k12 + LOOK) % NIDX

                @pl.when(ch + 2 * LOOK < NCHUNK)
                def _():
                    fire_idx(ch + 2 * LOOK, (k12 + 2 * LOOK) % NIDX)

                @pl.when(ch + LOOK < NCHUNK)
                def _():
                    wait_idx(ch + LOOK, k4)

                    @pl.when(ch >= 2)
                    def _():
                        wait_scatter(r4, (k12 + NIDX - 2) % NIDX)
                    fire_gather(r4, k4)

                wait_gather(r, k)
                compute(r)
                fire_scatter(r, k)
            return 0
        lax.fori_loop(0, NCHUNK // 12, block_body, 0)

        for cc in range(NCHUNK - NROT, NCHUNK):
            wait_scatter(cc % NROT, cc % NIDX)

        plsc.subcore_barrier()
        pltpu.sync_copy(acc.at[pl.ds(r0, ROWS_PER_TILE)],
                        out_hbm.at[cid, pl.ds(r0, ROWS_PER_TILE)])

    return edge_kernel


def _bn(h, g, b):
    mu = jnp.mean(h, axis=0, keepdims=True)
    var = jnp.mean((h - mu) ** 2, axis=0, keepdims=True)
    return g[None, :] * (h - mu) / jnp.sqrt(var + 1e-5) + b[None, :]


def _elu(h):
    return jnp.where(h > 0, h, jnp.exp(jnp.minimum(h, 0.0)) - 1.0)


def _stage_a_body(x_ref, wp_ref, bp_ref, g1_ref, be1_ref, w1_ref, as1_ref,
                  ad1_ref, hp_ref, t1_ref, adt1_ref):
    x = x_ref[...]
    h0 = jnp.dot(x, wp_ref[...], preferred_element_type=jnp.float32)
    h0 = h0 + bp_ref[...][None, :]
    hp = _elu(_bn(h0, g1_ref[...], be1_ref[...]))
    hp_ref[...] = hp
    h1 = jnp.dot(hp, w1_ref[...], preferred_element_type=jnp.float32)
    m = _head_expand_mask(NHEADS1, 128 // NHEADS1)
    a_s = as1_ref[...]
    a_d = ad1_ref[...]
    asrc = jnp.dot(h1, (m * a_s[None, :]).T, preferred_element_type=jnp.float32)
    adst = jnp.dot(h1, (m * a_d[None, :]).T, preferred_element_type=jnp.float32)
    zpadN = jnp.zeros((N, TCOLS - 136), jnp.float32)
    body = jnp.concatenate([h1, asrc, zpadN], axis=1)
    sent = jnp.concatenate([
        jnp.zeros((NPAD - N, 128), jnp.float32),
        jnp.full((NPAD - N, 8), _SENT, jnp.float32),
        jnp.zeros((NPAD - N, TCOLS - 136), jnp.float32)], axis=1)
    t1_ref[...] = jnp.concatenate([body, sent], axis=0)
    adt = jnp.concatenate([adst, jnp.zeros((N, ACOLS - 8), jnp.float32)], axis=1)
    adt1_ref[...] = jnp.concatenate(
        [adt, jnp.zeros((NPAD - N, ACOLS), jnp.float32)], axis=0)


def _stage_c_body(acc_ref, hp_ref, g2_ref, be2_ref, bc1_ref, w2_ref, as2_ref,
                  ad2_ref, h2_ref, t2_ref, adt2_ref):
    s = acc_ref[0] + acc_ref[1]
    num = s[0:N, 0:128]
    den8 = s[0:N, 128:136]
    m = _head_expand_mask(NHEADS1, 128 // NHEADS1)
    denf = jnp.dot(den8, m, preferred_element_type=jnp.float32)
    o1 = num / (denf + 1e-16) + bc1_ref[...][None, :]
    h2 = _elu(_bn(o1, g2_ref[...], be2_ref[...]) + hp_ref[...])
    h2_ref[...] = h2
    h2w = jnp.dot(h2, w2_ref[...], preferred_element_type=jnp.float32)
    a_s = as2_ref[...]
    a_d = ad2_ref[...]
    asrc = jnp.dot(h2w, a_s[:, None], preferred_element_type=jnp.float32)
    adst = jnp.dot(h2w, a_d[:, None], preferred_element_type=jnp.float32)
    body = jnp.concatenate(
        [h2w, asrc, jnp.zeros((N, TCOLS - 129), jnp.float32)], axis=1)
    sent = jnp.concatenate([
        jnp.zeros((NPAD - N, 128), jnp.float32),
        jnp.full((NPAD - N, 1), _SENT, jnp.float32),
        jnp.zeros((NPAD - N, TCOLS - 129), jnp.float32)], axis=1)
    t2_ref[...] = jnp.concatenate([body, sent], axis=0)
    adt = jnp.concatenate([adst, jnp.zeros((N, ACOLS - 1), jnp.float32)], axis=1)
    adt2_ref[...] = jnp.concatenate(
        [adt, jnp.zeros((NPAD - N, ACOLS), jnp.float32)], axis=0)


def _stage_e_body(acc_ref, h2_ref, g3_ref, be3_ref, bc2_ref, wc_ref, bcls_ref,
                  out_ref):
    s = acc_ref[0] + acc_ref[1]
    num = s[0:N, 0:128]
    den = s[0:N, 128:129]
    o2 = num / (den + 1e-16) + bc2_ref[...][None, :]
    h3 = _elu(_bn(o2, g3_ref[...], be3_ref[...]) + h2_ref[...])
    out_ref[...] = jnp.dot(h3, wc_ref[...], preferred_element_type=jnp.float32) \
        + bcls_ref[...][None, :]


_stage_a = pl.pallas_call(
    _stage_a_body,
    out_shape=[
        jax.ShapeDtypeStruct((N, D), jnp.float32),
        jax.ShapeDtypeStruct((NPAD, TCOLS), jnp.float32),
        jax.ShapeDtypeStruct((NPAD, ACOLS), jnp.float32),
    ],
)

_stage_c = pl.pallas_call(
    _stage_c_body,
    out_shape=[
        jax.ShapeDtypeStruct((N, D), jnp.float32),
        jax.ShapeDtypeStruct((NPAD, TCOLS), jnp.float32),
        jax.ShapeDtypeStruct((NPAD, ACOLS), jnp.float32),
    ],
)

_stage_e = pl.pallas_call(
    _stage_e_body,
    out_shape=jax.ShapeDtypeStruct((N, 40), jnp.float32),
)


def kernel(x, edge_index, W_proj, b_proj, g1, be1, W1, as1, ad1, bc1,
           g2, be2, W2, as2, ad2, bc2, g3, be3, W_cls, b_cls):
    ei = edge_index.astype(jnp.int32)
    loop = jnp.arange(N, dtype=jnp.int32)
    npad_e = EPAD - ETOT
    padi = jnp.full((npad_e,), N, jnp.int32)
    padd = N + (jnp.arange(npad_e, dtype=jnp.int32) % (NPAD - N))
    src = jnp.concatenate([ei[0], loop, padi])
    dst = jnp.concatenate([ei[1], loop, padd])
    ed = jnp.stack([src, dst])

    hp, t1, adt1 = _stage_a(x, W_proj, b_proj, g1, be1, W1,
                            as1.reshape(-1), ad1.reshape(-1))
    acc1 = _make_edge_kernel(NHEADS1)(t1, adt1, ed)
    h2, t2, adt2 = _stage_c(acc1, hp, g2, be2, bc1, W2,
                            as2.reshape(-1), ad2.reshape(-1))
    acc2 = _make_edge_kernel(1)(t2, adt2, ed)
    return _stage_e(acc2, h2, g3, be3, bc2, W_cls, b_cls)


# speedup vs baseline: 1.9749x; 1.9749x over previous
import functools

import jax
import jax.numpy as jnp
from jax import lax
from jax.experimental import pallas as pl
from jax.experimental.pallas import tpu as pltpu
from jax.experimental.pallas import tpu_sc as plsc

N = 10000
D = 128
NHEADS1 = 8
E = 320000
ETOT = E + N

NC = 2
NS = 16
NW = NC * NS

NPAD = 10112
ROWS_PER_TILE = NPAD // NS

FEAT = 64
TCOLS = 72
ACOLS = 16

C = 48
NCHUNK = 216
PER_W = C * NCHUNK
EPAD = PER_W * NW
NROT = 6
NIDX = 12
LOOK = 4

_SENT = -1e30


def _head_expand_mask(heads, oc):
    r = lax.broadcasted_iota(jnp.int32, (heads, 128), 0)
    c = lax.broadcasted_iota(jnp.int32, (heads, 128), 1) // oc
    return (r == c).astype(jnp.float32)


@functools.lru_cache(maxsize=None)
def _make_edge_kernel(heads, aofs):
    oc = FEAT // heads
    mesh = plsc.VectorSubcoreMesh(core_axis_name="c", subcore_axis_name="s",
                                  num_cores=NC, num_subcores=NS)

    @functools.partial(
        pl.kernel,
        mesh=mesh,
        compiler_params=pltpu.CompilerParams(use_tc_tiling_on_sc=False,
                                             needs_layout_passes=False),
        out_type=jax.ShapeDtypeStruct((NC, NPAD, TCOLS), jnp.float32),
        scratch_types=[
            pltpu.VMEM((NIDX, 2, C), jnp.int32),
            pltpu.VMEM((NROT, C, TCOLS), jnp.float32),
            pltpu.VMEM((NROT, C, ACOLS), jnp.float32),
            pltpu.VMEM_SHARED((NPAD, TCOLS), jnp.float32),
            pltpu.VMEM_SHARED((NPAD, ACOLS), jnp.float32),
            pltpu.VMEM_SHARED((NPAD, TCOLS), jnp.float32),
            pltpu.SemaphoreType.DMA((NIDX,)),
            pltpu.SemaphoreType.DMA((NROT,)),
            pltpu.SemaphoreType.DMA((NROT,)),
        ],
    )
    def edge_kernel(tab_hbm, adst_hbm, ed_hbm, out_hbm,
                    edb, rows, adstb, tspm, aspm, acc, semi, semg, sems):
        cid = lax.axis_index("c")
        sid = lax.axis_index("s")
        wid = cid * NS + sid

        r0 = sid * ROWS_PER_TILE
        pltpu.sync_copy(tab_hbm.at[pl.ds(r0, ROWS_PER_TILE)],
                        tspm.at[pl.ds(r0, ROWS_PER_TILE)])
        pltpu.sync_copy(adst_hbm.at[pl.ds(r0, ROWS_PER_TILE)],
                        aspm.at[pl.ds(r0, ROWS_PER_TILE)])

        def _zrow(i, _):
            for off in (0, 16, 32, 48, TCOLS - 16):
                rows[0, i, pl.ds(off, 16)] = jnp.zeros((16,), jnp.float32)
            return 0
        lax.fori_loop(0, C, _zrow, 0)
        for off in range(0, ROWS_PER_TILE, C):
            nrow = min(C, ROWS_PER_TILE - off)
            pltpu.sync_copy(rows.at[0, pl.ds(0, nrow)],
                            acc.at[pl.ds(r0 + off, nrow)])
        plsc.subcore_barrier()

        iota16 = lax.iota(jnp.int32, 16)
        ebase = wid * PER_W

        def fire_idx(ch, k):
            pltpu.async_copy(ed_hbm.at[:, pl.ds(ebase + ch * C, C)],
                             edb.at[k], semi.at[k])

        def wait_idx(ch, k):
            pltpu.make_async_copy(ed_hbm.at[:, pl.ds(ebase + ch * C, C)],
                                  edb.at[k], semi.at[k]).wait()

        def fire_gather(r, k):
            pltpu.async_copy(tspm.at[edb.at[k, 0]], rows.at[r], semg.at[r])
            pltpu.async_copy(aspm.at[edb.at[k, 1]], adstb.at[r], semg.at[r])

        def wait_gather(r, k):
            pltpu.make_async_copy(tspm.at[edb.at[k, 0]], rows.at[r],
                                  semg.at[r]).wait()
            pltpu.make_async_copy(aspm.at[edb.at[k, 1]], adstb.at[r],
                                  semg.at[r]).wait()

        def fire_scatter(r, k):
            pltpu.async_copy(rows.at[r], acc.at[edb.at[k, 1]], sems.at[r],
                             add=True)

        def wait_scatter(r, k):
            pltpu.make_async_copy(rows.at[r], acc.at[edb.at[k, 1]],
                                  sems.at[r]).wait()

        def compute(r):
            @plsc.parallel_loop(0, C // 16, unroll=2)
            def jbody(j):
                rb = j * 16 + iota16
                for h in range(heads):
                    colv = jnp.full((16,), FEAT + h, jnp.int32)
                    av = plsc.load_gather(rows.at[r], [rb, colv])
                    dv = plsc.load_gather(
                        adstb.at[r],
                        [rb, jnp.full((16,), aofs + h, jnp.int32)])
                    a = av + dv
                    a = jnp.maximum(a, a * jnp.float32(0.2))
                    plsc.store_scatter(rows.at[r], [rb, colv], jnp.exp(a))

            @plsc.parallel_loop(0, C, unroll=4)
            def ebody(e):
                exv = rows[r, e, pl.ds(FEAT - 8, 16)]
                for h in range(heads):
                    s = exv[8 + h]
                    for cc in range(oc // 16):
                        col = h * oc + cc * 16
                        rows[r, e, pl.ds(col, 16)] = \
                            rows[r, e, pl.ds(col, 16)] * s

        for i in range(2 * LOOK):
            fire_idx(i, i % NIDX)
        for i in range(LOOK):
            wait_idx(i, i % NIDX)
            fire_gather(i % NROT, i % NIDX)

        def block_body(t, _):
            ch0 = t * 12
            for k12 in range(12):
                ch = ch0 + k12
                r = k12 % NROT
                k = k12 % NIDX
                r4 = (k12 + LOOK) % NROT
                k4 = (k12 + LOOK) % NIDX

                @pl.when(ch + 2 * LOOK < NCHUNK)
                def _():
                    fire_idx(ch + 2 * LOOK, (k12 + 2 * LOOK) % NIDX)

                @pl.when(ch + LOOK < NCHUNK)
                def _():
                    wait_idx(ch + LOOK, k4)

                    @pl.when(ch >= 2)
                    def _():
                        wait_scatter(r4, (k12 + NIDX - 2) % NIDX)
                    fire_gather(r4, k4)

                wait_gather(r, k)
                compute(r)
                fire_scatter(r, k)
            return 0
        lax.fori_loop(0, NCHUNK // 12, block_body, 0)

        for cc in range(NCHUNK - NROT, NCHUNK):
            wait_scatter(cc % NROT, cc % NIDX)

        plsc.subcore_barrier()
        pltpu.sync_copy(acc.at[pl.ds(r0, ROWS_PER_TILE)],
                        out_hbm.at[cid, pl.ds(r0, ROWS_PER_TILE)])

    return edge_kernel


def _bn(h, g, b):
    mu = jnp.mean(h, axis=0, keepdims=True)
    var = jnp.mean((h - mu) ** 2, axis=0, keepdims=True)
    return g[None, :] * (h - mu) / jnp.sqrt(var + 1e-5) + b[None, :]


def _elu(h):
    return jnp.where(h > 0, h, jnp.exp(jnp.minimum(h, 0.0)) - 1.0)


def _pack_tables(hw, asrc, adst, nh):
    nph = nh // 2
    zpad = jnp.zeros((N, TCOLS - FEAT - nph), jnp.float32)
    sent = jnp.concatenate([
        jnp.zeros((NPAD - N, FEAT), jnp.float32),
        jnp.full((NPAD - N, TCOLS - FEAT), _SENT, jnp.float32)], axis=1)
    ta = jnp.concatenate([
        jnp.concatenate([hw[:, 0:FEAT], asrc[:, 0:nph], zpad], axis=1),
        sent], axis=0)
    tb = jnp.concatenate([
        jnp.concatenate([hw[:, FEAT:2 * FEAT], asrc[:, nph:nh], zpad],
                        axis=1),
        sent], axis=0)
    adt = jnp.concatenate(
        [adst, jnp.zeros((N, ACOLS - adst.shape[1]), jnp.float32)], axis=1)
    adt = jnp.concatenate(
        [adt, jnp.zeros((NPAD - N, ACOLS), jnp.float32)], axis=0)
    return ta, tb, adt


def _stage_a_body(x_ref, wp_ref, bp_ref, g1_ref, be1_ref, w1_ref, as1_ref,
                  ad1_ref, hp_ref, t1a_ref, t1b_ref, adt1_ref):
    x = x_ref[...]
    h0 = jnp.dot(x, wp_ref[...], preferred_element_type=jnp.float32)
    h0 = h0 + bp_ref[...][None, :]
    hp = _elu(_bn(h0, g1_ref[...], be1_ref[...]))
    hp_ref[...] = hp
    h1 = jnp.dot(hp, w1_ref[...], preferred_element_type=jnp.float32)
    m = _head_expand_mask(NHEADS1, 128 // NHEADS1)
    a_s = as1_ref[...]
    a_d = ad1_ref[...]
    asrc = jnp.dot(h1, (m * a_s[None, :]).T, preferred_element_type=jnp.float32)
    adst = jnp.dot(h1, (m * a_d[None, :]).T, preferred_element_type=jnp.float32)
    ta, tb, adt = _pack_tables(h1, asrc, adst, NHEADS1)
    t1a_ref[...] = ta
    t1b_ref[...] = tb
    adt1_ref[...] = adt


def _stage_c_body(acca_ref, accb_ref, hp_ref, g2_ref, be2_ref, bc1_ref,
                  w2_ref, as2_ref, ad2_ref, h2_ref, t2a_ref, t2b_ref,
                  adt2_ref):
    sa = acca_ref[0] + acca_ref[1]
    sb = accb_ref[0] + accb_ref[1]
    num = jnp.concatenate([sa[0:N, 0:FEAT], sb[0:N, 0:FEAT]], axis=1)
    den8 = jnp.concatenate([sa[0:N, FEAT:FEAT + 4],
                            sb[0:N, FEAT:FEAT + 4]], axis=1)
    m = _head_expand_mask(NHEADS1, 128 // NHEADS1)
    denf = jnp.dot(den8, m, preferred_element_type=jnp.float32)
    o1 = num / (denf + 1e-16) + bc1_ref[...][None, :]
    h2 = _elu(_bn(o1, g2_ref[...], be2_ref[...]) + hp_ref[...])
    h2_ref[...] = h2
    h2w = jnp.dot(h2, w2_ref[...], preferred_element_type=jnp.float32)
    a_s = as2_ref[...]
    a_d = ad2_ref[...]
    asrc = jnp.dot(h2w, a_s[:, None], preferred_element_type=jnp.float32)
    adst = jnp.dot(h2w, a_d[:, None], preferred_element_type=jnp.float32)
    asrc2 = jnp.concatenate([asrc, asrc], axis=1)
    ta, tb, adt = _pack_tables(h2w, asrc2, adst, 2)
    t2a_ref[...] = ta
    t2b_ref[...] = tb
    adt2_ref[...] = adt


def _stage_e_body(acca_ref, accb_ref, h2_ref, g3_ref, be3_ref, bc2_ref,
                  wc_ref, bcls_ref, out_ref):
    sa = acca_ref[0] + acca_ref[1]
    sb = accb_ref[0] + accb_ref[1]
    num = jnp.concatenate([sa[0:N, 0:FEAT], sb[0:N, 0:FEAT]], axis=1)
    den = sa[0:N, FEAT:FEAT + 1]
    o2 = num / (den + 1e-16) + bc2_ref[...][None, :]
    h3 = _elu(_bn(o2, g3_ref[...], be3_ref[...]) + h2_ref[...])
    out_ref[...] = jnp.dot(h3, wc_ref[...], preferred_element_type=jnp.float32) \
        + bcls_ref[...][None, :]


_TC_PARAMS = pltpu.CompilerParams(vmem_limit_bytes=100 * 1024 * 1024)

_stage_a = pl.pallas_call(
    _stage_a_body,
    compiler_params=_TC_PARAMS,
    out_shape=[
        jax.ShapeDtypeStruct((N, D), jnp.float32),
        jax.ShapeDtypeStruct((NPAD, TCOLS), jnp.float32),
        jax.ShapeDtypeStruct((NPAD, TCOLS), jnp.float32),
        jax.ShapeDtypeStruct((NPAD, ACOLS), jnp.float32),
    ],
)

_stage_c = pl.pallas_call(
    _stage_c_body,
    compiler_params=_TC_PARAMS,
    out_shape=[
        jax.ShapeDtypeStruct((N, D), jnp.float32),
        jax.ShapeDtypeStruct((NPAD, TCOLS), jnp.float32),
        jax.ShapeDtypeStruct((NPAD, TCOLS), jnp.float32),
        jax.ShapeDtypeStruct((NPAD, ACOLS), jnp.float32),
    ],
)

_stage_e = pl.pallas_call(
    _stage_e_body,
    compiler_params=_TC_PARAMS,
    out_shape=jax.ShapeDtypeStruct((N, 40), jnp.float32),
)


def kernel(x, edge_index, W_proj, b_proj, g1, be1, W1, as1, ad1, bc1,
           g2, be2, W2, as2, ad2, bc2, g3, be3, W_cls, b_cls):
    ei = edge_index.astype(jnp.int32)
    loop = jnp.arange(N, dtype=jnp.int32)
    npad_e = EPAD - ETOT
    padi = jnp.full((npad_e,), N, jnp.int32)
    padd = N + (jnp.arange(npad_e, dtype=jnp.int32) % (NPAD - N))
    src = jnp.concatenate([ei[0], loop, padi])
    dst = jnp.concatenate([ei[1], loop, padd])
    ed = jnp.stack([src, dst])

    hp, t1a, t1b, adt1 = _stage_a(x, W_proj, b_proj, g1, be1, W1,
                                  as1.reshape(-1), ad1.reshape(-1))
    acc1a = _make_edge_kernel(4, 0)(t1a, adt1, ed)
    acc1b = _make_edge_kernel(4, 4)(t1b, adt1, ed)
    h2, t2a, t2b, adt2 = _stage_c(acc1a, acc1b, hp, g2, be2, bc1, W2,
                                  as2.reshape(-1), ad2.reshape(-1))
    acc2a = _make_edge_kernel(1, 0)(t2a, adt2, ed)
    acc2b = _make_edge_kernel(1, 0)(t2b, adt2, ed)
    return _stage_e(acc2a, acc2b, h2, g3, be3, bc2, W_cls, b_cls)


# speedup vs baseline: 2.0516x; 1.0389x over previous
import functools

import jax
import jax.numpy as jnp
from jax import lax
from jax.experimental import pallas as pl
from jax.experimental.pallas import tpu as pltpu
from jax.experimental.pallas import tpu_sc as plsc

N = 10000
D = 128
NHEADS1 = 8
E = 320000
ETOT = E + N

NC = 2
NS = 16
NW = NC * NS

NPAD = 10112
ROWS_PER_TILE = NPAD // NS

FEAT = 64
TCOLS = 72
ACOLS = 16

C = 48
NCHUNK = 432
PER_W = C * NCHUNK
EPAD = PER_W * NS
NROT = 6
NIDX = 12
LOOK = 4

_SENT = -1e30


def _head_expand_mask(heads, oc):
    r = lax.broadcasted_iota(jnp.int32, (heads, 128), 0)
    c = lax.broadcasted_iota(jnp.int32, (heads, 128), 1) // oc
    return (r == c).astype(jnp.float32)


@functools.lru_cache(maxsize=None)
def _make_edge_kernel(heads, dyn_aofs):
    oc = FEAT // heads
    mesh = plsc.VectorSubcoreMesh(core_axis_name="c", subcore_axis_name="s",
                                  num_cores=NC, num_subcores=NS)

    @functools.partial(
        pl.kernel,
        mesh=mesh,
        compiler_params=pltpu.CompilerParams(use_tc_tiling_on_sc=False,
                                             needs_layout_passes=False),
        out_type=jax.ShapeDtypeStruct((NC, NPAD, TCOLS), jnp.float32),
        scratch_types=[
            pltpu.VMEM((NIDX, 2, C), jnp.int32),
            pltpu.VMEM((NROT, C, TCOLS), jnp.float32),
            pltpu.VMEM((NROT, C, ACOLS), jnp.float32),
            pltpu.VMEM_SHARED((NPAD, TCOLS), jnp.float32),
            pltpu.VMEM_SHARED((NPAD, ACOLS), jnp.float32),
            pltpu.VMEM_SHARED((NPAD, TCOLS), jnp.float32),
            pltpu.SemaphoreType.DMA((NIDX,)),
            pltpu.SemaphoreType.DMA((NROT,)),
            pltpu.SemaphoreType.DMA((NROT,)),
        ],
    )
    def edge_kernel(tab_hbm, adst_hbm, ed_hbm, out_hbm,
                    edb, rows, adstb, tspm, aspm, acc, semi, semg, sems):
        cid = lax.axis_index("c")
        sid = lax.axis_index("s")

        r0 = sid * ROWS_PER_TILE
        pltpu.sync_copy(tab_hbm.at[cid, pl.ds(r0, ROWS_PER_TILE)],
                        tspm.at[pl.ds(r0, ROWS_PER_TILE)])
        pltpu.sync_copy(adst_hbm.at[pl.ds(r0, ROWS_PER_TILE)],
                        aspm.at[pl.ds(r0, ROWS_PER_TILE)])

        def _zrow(i, _):
            for off in (0, 16, 32, 48, TCOLS - 16):
                rows[0, i, pl.ds(off, 16)] = jnp.zeros((16,), jnp.float32)
            return 0
        lax.fori_loop(0, C, _zrow, 0)
        for off in range(0, ROWS_PER_TILE, C):
            nrow = min(C, ROWS_PER_TILE - off)
            pltpu.sync_copy(rows.at[0, pl.ds(0, nrow)],
                            acc.at[pl.ds(r0 + off, nrow)])
        plsc.subcore_barrier()

        iota16 = lax.iota(jnp.int32, 16)
        ebase = sid * PER_W
        aofs = cid * heads if dyn_aofs else 0

        def fire_idx(ch, k):
            pltpu.async_copy(ed_hbm.at[:, pl.ds(ebase + ch * C, C)],
                             edb.at[k], semi.at[k])

        def wait_idx(ch, k):
            pltpu.make_async_copy(ed_hbm.at[:, pl.ds(ebase + ch * C, C)],
                                  edb.at[k], semi.at[k]).wait()

        def fire_gather(r, k):
            pltpu.async_copy(tspm.at[edb.at[k, 0]], rows.at[r], semg.at[r])
            pltpu.async_copy(aspm.at[edb.at[k, 1]], adstb.at[r], semg.at[r])

        def wait_gather(r, k):
            pltpu.make_async_copy(tspm.at[edb.at[k, 0]], rows.at[r],
                                  semg.at[r]).wait()
            pltpu.make_async_copy(aspm.at[edb.at[k, 1]], adstb.at[r],
                                  semg.at[r]).wait()

        def fire_scatter(r, k):
            pltpu.async_copy(rows.at[r], acc.at[edb.at[k, 1]], sems.at[r],
                             add=True)

        def wait_scatter(r, k):
            pltpu.make_async_copy(rows.at[r], acc.at[edb.at[k, 1]],
                                  sems.at[r]).wait()

        def compute(r):
            @plsc.parallel_loop(0, C // 16, unroll=2)
            def jbody(j):
                rb = j * 16 + iota16
                for h in range(heads):
                    colv = jnp.full((16,), FEAT + h, jnp.int32)
                    av = plsc.load_gather(rows.at[r], [rb, colv])
                    dv = plsc.load_gather(
                        adstb.at[r],
                        [rb, jnp.broadcast_to(aofs + h, (16,)).astype(jnp.int32)])
                    a = av + dv
                    a = jnp.maximum(a, a * jnp.float32(0.2))
                    plsc.store_scatter(rows.at[r], [rb, colv], jnp.exp(a))

            @plsc.parallel_loop(0, C, unroll=4)
            def ebody(e):
                exv = rows[r, e, pl.ds(FEAT - 8, 16)]
                for h in range(heads):
                    s = exv[8 + h]
                    for cc in range(oc // 16):
                        col = h * oc + cc * 16
                        rows[r, e, pl.ds(col, 16)] = \
                            rows[r, e, pl.ds(col, 16)] * s

        for i in range(2 * LOOK):
            fire_idx(i, i % NIDX)
        for i in range(LOOK):
            wait_idx(i, i % NIDX)
            fire_gather(i % NROT, i % NIDX)

        def block_body(t, _):
            ch0 = t * 12
            for k12 in range(12):
                ch = ch0 + k12
                r = k12 % NROT
                k = k12 % NIDX
                r4 = (k12 + LOOK) % NROT
                k4 = (k12 + LOOK) % NIDX

                @pl.when(ch + 2 * LOOK < NCHUNK)
                def _():
                    fire_idx(ch + 2 * LOOK, (k12 + 2 * LOOK) % NIDX)

                @pl.when(ch + LOOK < NCHUNK)
                def _():
                    wait_idx(ch + LOOK, k4)

                    @pl.when(ch >= 2)
                    def _():
                        wait_scatter(r4, (k12 + NIDX - 2) % NIDX)
                    fire_gather(r4, k4)

                wait_gather(r, k)
                compute(r)
                fire_scatter(r, k)
            return 0
        lax.fori_loop(0, NCHUNK // 12, block_body, 0)

        for cc in range(NCHUNK - NROT, NCHUNK):
            wait_scatter(cc % NROT, cc % NIDX)

        plsc.subcore_barrier()
        pltpu.sync_copy(acc.at[pl.ds(r0, ROWS_PER_TILE)],
                        out_hbm.at[cid, pl.ds(r0, ROWS_PER_TILE)])

    return edge_kernel


def _bn(h, g, b):
    mu = jnp.mean(h, axis=0, keepdims=True)
    var = jnp.mean((h - mu) ** 2, axis=0, keepdims=True)
    return g[None, :] * (h - mu) / jnp.sqrt(var + 1e-5) + b[None, :]


def _elu(h):
    return jnp.where(h > 0, h, jnp.exp(jnp.minimum(h, 0.0)) - 1.0)


def _pack_tables(hw, asrc, adst, nh):
    nph = nh // 2
    zpad = jnp.zeros((N, TCOLS - FEAT - nph), jnp.float32)
    sent = jnp.concatenate([
        jnp.zeros((NPAD - N, FEAT), jnp.float32),
        jnp.full((NPAD - N, TCOLS - FEAT), _SENT, jnp.float32)], axis=1)
    ta = jnp.concatenate([
        jnp.concatenate([hw[:, 0:FEAT], asrc[:, 0:nph], zpad], axis=1),
        sent], axis=0)
    tb = jnp.concatenate([
        jnp.concatenate([hw[:, FEAT:2 * FEAT], asrc[:, nph:nh], zpad],
                        axis=1),
        sent], axis=0)
    adt = jnp.concatenate(
        [adst, jnp.zeros((N, ACOLS - adst.shape[1]), jnp.float32)], axis=1)
    adt = jnp.concatenate(
        [adt, jnp.zeros((NPAD - N, ACOLS), jnp.float32)], axis=0)
    return ta, tb, adt


def _stage_a_body(x_ref, wp_ref, bp_ref, g1_ref, be1_ref, w1_ref, as1_ref,
                  ad1_ref, hp_ref, t1a_ref, t1b_ref, adt1_ref):
    x = x_ref[...]
    h0 = jnp.dot(x, wp_ref[...], preferred_element_type=jnp.float32)
    h0 = h0 + bp_ref[...][None, :]
    hp = _elu(_bn(h0, g1_ref[...], be1_ref[...]))
    hp_ref[...] = hp
    h1 = jnp.dot(hp, w1_ref[...], preferred_element_type=jnp.float32)
    m = _head_expand_mask(NHEADS1, 128 // NHEADS1)
    a_s = as1_ref[...]
    a_d = ad1_ref[...]
    asrc = jnp.dot(h1, (m * a_s[None, :]).T, preferred_element_type=jnp.float32)
    adst = jnp.dot(h1, (m * a_d[None, :]).T, preferred_element_type=jnp.float32)
    ta, tb, adt = _pack_tables(h1, asrc, adst, NHEADS1)
    t1a_ref[...] = ta
    t1b_ref[...] = tb
    adt1_ref[...] = adt


def _stage_c_body(acc_ref, hp_ref, g2_ref, be2_ref, bc1_ref,
                  w2_ref, as2_ref, ad2_ref, h2_ref, t2a_ref, t2b_ref,
                  adt2_ref):
    sa = acc_ref[0]
    sb = acc_ref[1]
    num = jnp.concatenate([sa[0:N, 0:FEAT], sb[0:N, 0:FEAT]], axis=1)
    den8 = jnp.concatenate([sa[0:N, FEAT:FEAT + 4],
                            sb[0:N, FEAT:FEAT + 4]], axis=1)
    m = _head_expand_mask(NHEADS1, 128 // NHEADS1)
    denf = jnp.dot(den8, m, preferred_element_type=jnp.float32)
    o1 = num / (denf + 1e-16) + bc1_ref[...][None, :]
    h2 = _elu(_bn(o1, g2_ref[...], be2_ref[...]) + hp_ref[...])
    h2_ref[...] = h2
    h2w = jnp.dot(h2, w2_ref[...], preferred_element_type=jnp.float32)
    a_s = as2_ref[...]
    a_d = ad2_ref[...]
    asrc = jnp.dot(h2w, a_s[:, None], preferred_element_type=jnp.float32)
    adst = jnp.dot(h2w, a_d[:, None], preferred_element_type=jnp.float32)
    asrc2 = jnp.concatenate([asrc, asrc], axis=1)
    ta, tb, adt = _pack_tables(h2w, asrc2, adst, 2)
    t2a_ref[...] = ta
    t2b_ref[...] = tb
    adt2_ref[...] = adt


def _stage_e_body(acc_ref, h2_ref, g3_ref, be3_ref, bc2_ref,
                  wc_ref, bcls_ref, out_ref):
    sa = acc_ref[0]
    sb = acc_ref[1]
    num = jnp.concatenate([sa[0:N, 0:FEAT], sb[0:N, 0:FEAT]], axis=1)
    den = sa[0:N, FEAT:FEAT + 1]
    o2 = num / (den + 1e-16) + bc2_ref[...][None, :]
    h3 = _elu(_bn(o2, g3_ref[...], be3_ref[...]) + h2_ref[...])
    out_ref[...] = jnp.dot(h3, wc_ref[...], preferred_element_type=jnp.float32) \
        + bcls_ref[...][None, :]


_TC_PARAMS = pltpu.CompilerParams(vmem_limit_bytes=100 * 1024 * 1024)

_stage_a = pl.pallas_call(
    _stage_a_body,
    compiler_params=_TC_PARAMS,
    out_shape=[
        jax.ShapeDtypeStruct((N, D), jnp.float32),
        jax.ShapeDtypeStruct((NPAD, TCOLS), jnp.float32),
        jax.ShapeDtypeStruct((NPAD, TCOLS), jnp.float32),
        jax.ShapeDtypeStruct((NPAD, ACOLS), jnp.float32),
    ],
)

_stage_c = pl.pallas_call(
    _stage_c_body,
    compiler_params=_TC_PARAMS,
    out_shape=[
        jax.ShapeDtypeStruct((N, D), jnp.float32),
        jax.ShapeDtypeStruct((NPAD, TCOLS), jnp.float32),
        jax.ShapeDtypeStruct((NPAD, TCOLS), jnp.float32),
        jax.ShapeDtypeStruct((NPAD, ACOLS), jnp.float32),
    ],
)

_stage_e = pl.pallas_call(
    _stage_e_body,
    compiler_params=_TC_PARAMS,
    out_shape=jax.ShapeDtypeStruct((N, 40), jnp.float32),
)


def kernel(x, edge_index, W_proj, b_proj, g1, be1, W1, as1, ad1, bc1,
           g2, be2, W2, as2, ad2, bc2, g3, be3, W_cls, b_cls):
    ei = edge_index.astype(jnp.int32)
    loop = jnp.arange(N, dtype=jnp.int32)
    npad_e = EPAD - ETOT
    padi = jnp.full((npad_e,), N, jnp.int32)
    padd = N + (jnp.arange(npad_e, dtype=jnp.int32) % (NPAD - N))
    src = jnp.concatenate([ei[0], loop, padi])
    dst = jnp.concatenate([ei[1], loop, padd])
    ed = jnp.stack([src, dst])

    hp, t1a, t1b, adt1 = _stage_a(x, W_proj, b_proj, g1, be1, W1,
                                  as1.reshape(-1), ad1.reshape(-1))
    acc1 = _make_edge_kernel(4, True)(jnp.stack([t1a, t1b]), adt1, ed)
    h2, t2a, t2b, adt2 = _stage_c(acc1, hp, g2, be2, bc1, W2,
                                  as2.reshape(-1), ad2.reshape(-1))
    acc2 = _make_edge_kernel(1, False)(jnp.stack([t2a, t2b]), adt2, ed)
    return _stage_e(acc2, h2, g3, be3, bc2, W_cls, b_cls)


# speedup vs baseline: 2.1133x; 1.0301x over previous
import functools

import jax
import jax.numpy as jnp
from jax import lax
from jax.experimental import pallas as pl
from jax.experimental.pallas import tpu as pltpu
from jax.experimental.pallas import tpu_sc as plsc

N = 10000
D = 128
NHEADS1 = 8
E = 320000
ETOT = E + N

NC = 2
NS = 16
NW = NC * NS

NPAD = 10112
ROWS_PER_TILE = NPAD // NS

FEAT = 64
TCOLS = 72
ACOLS = 8

C = 48
NCHUNK = 432
PER_W = C * NCHUNK
EPAD = PER_W * NS
NROT = 6
NIDX = 12
LOOK = 4

_SENT = -1e30


def _head_expand_mask(heads, oc):
    r = lax.broadcasted_iota(jnp.int32, (heads, 128), 0)
    c = lax.broadcasted_iota(jnp.int32, (heads, 128), 1) // oc
    return (r == c).astype(jnp.float32)


@functools.lru_cache(maxsize=None)
def _make_edge_kernel(heads):
    oc = FEAT // heads
    mesh = plsc.VectorSubcoreMesh(core_axis_name="c", subcore_axis_name="s",
                                  num_cores=NC, num_subcores=NS)

    @functools.partial(
        pl.kernel,
        mesh=mesh,
        compiler_params=pltpu.CompilerParams(use_tc_tiling_on_sc=False,
                                             needs_layout_passes=False),
        out_type=jax.ShapeDtypeStruct((NC, NPAD, TCOLS), jnp.float32),
        scratch_types=[
            pltpu.VMEM((NIDX, 2, C), jnp.int32),
            pltpu.VMEM((NROT, C, TCOLS), jnp.float32),
            pltpu.VMEM((NROT, C, ACOLS), jnp.float32),
            pltpu.VMEM_SHARED((NPAD, TCOLS), jnp.float32),
            pltpu.VMEM_SHARED((NPAD, ACOLS), jnp.float32),
            pltpu.VMEM_SHARED((NPAD, TCOLS), jnp.float32),
            pltpu.SemaphoreType.DMA((NIDX,)),
            pltpu.SemaphoreType.DMA((NROT,)),
            pltpu.SemaphoreType.DMA((NROT,)),
        ],
    )
    def edge_kernel(tab_hbm, adst_hbm, ed_hbm, out_hbm,
                    edb, rows, adstb, tspm, aspm, acc, semi, semg, sems):
        cid = lax.axis_index("c")
        sid = lax.axis_index("s")

        r0 = sid * ROWS_PER_TILE
        pltpu.sync_copy(tab_hbm.at[cid, pl.ds(r0, ROWS_PER_TILE)],
                        tspm.at[pl.ds(r0, ROWS_PER_TILE)])
        pltpu.sync_copy(adst_hbm.at[cid, pl.ds(r0, ROWS_PER_TILE)],
                        aspm.at[pl.ds(r0, ROWS_PER_TILE)])

        def _zrow(i, _):
            for off in (0, 16, 32, 48, TCOLS - 16):
                rows[0, i, pl.ds(off, 16)] = jnp.zeros((16,), jnp.float32)
            return 0
        lax.fori_loop(0, C, _zrow, 0)
        for off in range(0, ROWS_PER_TILE, C):
            nrow = min(C, ROWS_PER_TILE - off)
            pltpu.sync_copy(rows.at[0, pl.ds(0, nrow)],
                            acc.at[pl.ds(r0 + off, nrow)])
        plsc.subcore_barrier()

        iota16 = lax.iota(jnp.int32, 16)
        ebase = sid * PER_W

        def fire_idx(ch, k):
            pltpu.async_copy(ed_hbm.at[:, pl.ds(ebase + ch * C, C)],
                             edb.at[k], semi.at[k])

        def wait_idx(ch, k):
            pltpu.make_async_copy(ed_hbm.at[:, pl.ds(ebase + ch * C, C)],
                                  edb.at[k], semi.at[k]).wait()

        def fire_gather(r, k):
            pltpu.async_copy(tspm.at[edb.at[k, 0]], rows.at[r], semg.at[r])
            pltpu.async_copy(aspm.at[edb.at[k, 1]], adstb.at[r], semg.at[r])

        def wait_gather(r, k):
            pltpu.make_async_copy(tspm.at[edb.at[k, 0]], rows.at[r],
                                  semg.at[r]).wait()
            pltpu.make_async_copy(aspm.at[edb.at[k, 1]], adstb.at[r],
                                  semg.at[r]).wait()

        def fire_scatter(r, k):
            pltpu.async_copy(rows.at[r], acc.at[edb.at[k, 1]], sems.at[r],
                             add=True)

        def wait_scatter(r, k):
            pltpu.make_async_copy(rows.at[r], acc.at[edb.at[k, 1]],
                                  sems.at[r]).wait()

        def compute(r):
            @plsc.parallel_loop(0, C // 16, unroll=2)
            def jbody(j):
                rb = j * 16 + iota16
                for h in range(heads):
                    colv = jnp.full((16,), FEAT + h, jnp.int32)
                    av = plsc.load_gather(rows.at[r], [rb, colv])
                    dv = plsc.load_gather(
                        adstb.at[r], [rb, jnp.full((16,), h, jnp.int32)])
                    a = av + dv
                    a = jnp.maximum(a, a * jnp.float32(0.2))
                    plsc.store_scatter(rows.at[r], [rb, colv], jnp.exp(a))

            @plsc.parallel_loop(0, C, unroll=4)
            def ebody(e):
                exv = rows[r, e, pl.ds(FEAT - 8, 16)]
                for h in range(heads):
                    s = exv[8 + h]
                    for cc in range(oc // 16):
                        col = h * oc + cc * 16
                        rows[r, e, pl.ds(col, 16)] = \
                            rows[r, e, pl.ds(col, 16)] * s

        for i in range(2 * LOOK):
            fire_idx(i, i % NIDX)
        for i in range(LOOK):
            wait_idx(i, i % NIDX)
            fire_gather(i % NROT, i % NIDX)

        def block_body(t, _):
            ch0 = t * 12
            for k12 in range(12):
                ch = ch0 + k12
                r = k12 % NROT
                k = k12 % NIDX
                r4 = (k12 + LOOK) % NROT
                k4 = (k12 + LOOK) % NIDX

                @pl.when(ch + 2 * LOOK < NCHUNK)
                def _():
                    fire_idx(ch + 2 * LOOK, (k12 + 2 * LOOK) % NIDX)

                @pl.when(ch + LOOK < NCHUNK)
                def _():
                    wait_idx(ch + LOOK, k4)

                    @pl.when(ch >= 2)
                    def _():
                        wait_scatter(r4, (k12 + NIDX - 2) % NIDX)
                    fire_gather(r4, k4)

                wait_gather(r, k)
                compute(r)
                fire_scatter(r, k)
            return 0
        lax.fori_loop(0, NCHUNK // 12, block_body, 0)

        for cc in range(NCHUNK - NROT, NCHUNK):
            wait_scatter(cc % NROT, cc % NIDX)

        plsc.subcore_barrier()
        pltpu.sync_copy(acc.at[pl.ds(r0, ROWS_PER_TILE)],
                        out_hbm.at[cid, pl.ds(r0, ROWS_PER_TILE)])

    return edge_kernel


def _bn(h, g, b):
    mu = jnp.mean(h, axis=0, keepdims=True)
    var = jnp.mean((h - mu) ** 2, axis=0, keepdims=True)
    return g[None, :] * (h - mu) / jnp.sqrt(var + 1e-5) + b[None, :]


def _elu(h):
    return jnp.where(h > 0, h, jnp.exp(jnp.minimum(h, 0.0)) - 1.0)


def _pack_tables(hw, asrc, adst_a, adst_b, nh):
    nph = nh // 2
    zpad = jnp.zeros((N, TCOLS - FEAT - nph), jnp.float32)
    sent = jnp.concatenate([
        jnp.zeros((NPAD - N, FEAT), jnp.float32),
        jnp.full((NPAD - N, TCOLS - FEAT), _SENT, jnp.float32)], axis=1)
    ta = jnp.concatenate([
        jnp.concatenate([hw[:, 0:FEAT], asrc[:, 0:nph], zpad], axis=1),
        sent], axis=0)
    tb = jnp.concatenate([
        jnp.concatenate([hw[:, FEAT:2 * FEAT], asrc[:, nph:nh], zpad],
                        axis=1),
        sent], axis=0)

    def pad_ad(ad):
        adt = jnp.concatenate(
            [ad, jnp.zeros((N, ACOLS - ad.shape[1]), jnp.float32)], axis=1)
        return jnp.concatenate(
            [adt, jnp.zeros((NPAD - N, ACOLS), jnp.float32)], axis=0)
    return ta, tb, pad_ad(adst_a), pad_ad(adst_b)


def _stage_a_body(x_ref, wp_ref, bp_ref, g1_ref, be1_ref, w1_ref, as1_ref,
                  ad1_ref, hp_ref, t1_ref, adt1_ref):
    x = x_ref[...]
    h0 = jnp.dot(x, wp_ref[...], preferred_element_type=jnp.float32)
    h0 = h0 + bp_ref[...][None, :]
    hp = _elu(_bn(h0, g1_ref[...], be1_ref[...]))
    hp_ref[...] = hp
    h1 = jnp.dot(hp, w1_ref[...], preferred_element_type=jnp.float32)
    m = _head_expand_mask(NHEADS1, 128 // NHEADS1)
    a_s = as1_ref[...]
    a_d = ad1_ref[...]
    asrc = jnp.dot(h1, (m * a_s[None, :]).T, preferred_element_type=jnp.float32)
    adst = jnp.dot(h1, (m * a_d[None, :]).T, preferred_element_type=jnp.float32)
    ta, tb, ada, adb = _pack_tables(h1, asrc, adst[:, 0:4], adst[:, 4:8],
                                    NHEADS1)
    t1_ref[0] = ta
    t1_ref[1] = tb
    adt1_ref[0] = ada
    adt1_ref[1] = adb


def _stage_c_body(acc_ref, hp_ref, g2_ref, be2_ref, bc1_ref,
                  w2_ref, as2_ref, ad2_ref, h2_ref, t2_ref, adt2_ref):
    sa = acc_ref[0]
    sb = acc_ref[1]
    num = jnp.concatenate([sa[0:N, 0:FEAT], sb[0:N, 0:FEAT]], axis=1)
    den8 = jnp.concatenate([sa[0:N, FEAT:FEAT + 4],
                            sb[0:N, FEAT:FEAT + 4]], axis=1)
    m = _head_expand_mask(NHEADS1, 128 // NHEADS1)
    denf = jnp.dot(den8, m, preferred_element_type=jnp.float32)
    o1 = num / (denf + 1e-16) + bc1_ref[...][None, :]
    h2 = _elu(_bn(o1, g2_ref[...], be2_ref[...]) + hp_ref[...])
    h2_ref[...] = h2
    h2w = jnp.dot(h2, w2_ref[...], preferred_element_type=jnp.float32)
    a_s = as2_ref[...]
    a_d = ad2_ref[...]
    asrc = jnp.dot(h2w, a_s[:, None], preferred_element_type=jnp.float32)
    adst = jnp.dot(h2w, a_d[:, None], preferred_element_type=jnp.float32)
    asrc2 = jnp.concatenate([asrc, asrc], axis=1)
    ta, tb, ada, adb = _pack_tables(h2w, asrc2, adst, adst, 2)
    t2_ref[0] = ta
    t2_ref[1] = tb
    adt2_ref[0] = ada
    adt2_ref[1] = adb


def _stage_e_body(acc_ref, h2_ref, g3_ref, be3_ref, bc2_ref,
                  wc_ref, bcls_ref, out_ref):
    sa = acc_ref[0]
    sb = acc_ref[1]
    num = jnp.concatenate([sa[0:N, 0:FEAT], sb[0:N, 0:FEAT]], axis=1)
    den = sa[0:N, FEAT:FEAT + 1]
    o2 = num / (den + 1e-16) + bc2_ref[...][None, :]
    h3 = _elu(_bn(o2, g3_ref[...], be3_ref[...]) + h2_ref[...])
    out_ref[...] = jnp.dot(h3, wc_ref[...], preferred_element_type=jnp.float32) \
        + bcls_ref[...][None, :]


_TC_PARAMS = pltpu.CompilerParams(vmem_limit_bytes=100 * 1024 * 1024)

_stage_a = pl.pallas_call(
    _stage_a_body,
    compiler_params=_TC_PARAMS,
    out_shape=[
        jax.ShapeDtypeStruct((N, D), jnp.float32),
        jax.ShapeDtypeStruct((2, NPAD, TCOLS), jnp.float32),
        jax.ShapeDtypeStruct((2, NPAD, ACOLS), jnp.float32),
    ],
)

_stage_c = pl.pallas_call(
    _stage_c_body,
    compiler_params=_TC_PARAMS,
    out_shape=[
        jax.ShapeDtypeStruct((N, D), jnp.float32),
        jax.ShapeDtypeStruct((2, NPAD, TCOLS), jnp.float32),
        jax.ShapeDtypeStruct((2, NPAD, ACOLS), jnp.float32),
    ],
)

_stage_e = pl.pallas_call(
    _stage_e_body,
    compiler_params=_TC_PARAMS,
    out_shape=jax.ShapeDtypeStruct((N, 40), jnp.float32),
)


def kernel(x, edge_index, W_proj, b_proj, g1, be1, W1, as1, ad1, bc1,
           g2, be2, W2, as2, ad2, bc2, g3, be3, W_cls, b_cls):
    ei = edge_index.astype(jnp.int32)
    loop = jnp.arange(N, dtype=jnp.int32)
    npad_e = EPAD - ETOT
    padi = jnp.full((npad_e,), N, jnp.int32)
    padd = N + (jnp.arange(npad_e, dtype=jnp.int32) % (NPAD - N))
    src = jnp.concatenate([ei[0], loop, padi])
    dst = jnp.concatenate([ei[1], loop, padd])
    ed = jnp.stack([src, dst])

    hp, t1, adt1 = _stage_a(x, W_proj, b_proj, g1, be1, W1,
                            as1.reshape(-1), ad1.reshape(-1))
    acc1 = _make_edge_kernel(4)(t1, adt1, ed)
    h2, t2, adt2 = _stage_c(acc1, hp, g2, be2, bc1, W2,
                            as2.reshape(-1), ad2.reshape(-1))
    acc2 = _make_edge_kernel(1)(t2, adt2, ed)
    return _stage_e(acc2, h2, g3, be3, bc2, W_cls, b_cls)
